# baseline (speedup 1.0000x reference)
import functools

import jax
import jax.numpy as jnp
from jax import lax
from jax.experimental import pallas as pl
from jax.experimental.pallas import tpu as pltpu
from jax.experimental.pallas import tpu_sc as plsc

N = 10000
E = 320000
NB = 16
D_IN = 128
D_H = 64

NC = 2
NS = 16
NW = NC * NS
EPT = E // NW
C = 80
NCH = EPT // C
ZR = 640
ZTAIL = N - (NS - 1) * ZR
ZB = 80
NBUF = 10
GD = 7
MAIN = (NCH - GD) // NBUF
ND = 10240

R = 1024
GRID = ND // R
DCH = ND // NS

BN_SCALE = float(1.0 / (1.0 + 1e-5) ** 0.5)

_mesh = plsc.VectorSubcoreMesh(core_axis_name="c", subcore_axis_name="s")
_sc_params = pltpu.CompilerParams(use_tc_tiling_on_sc=False)
_sc_params_nl = pltpu.CompilerParams(use_tc_tiling_on_sc=False,
                                     needs_layout_passes=False)


def _deg_body(dst_hbm, w_hbm, out_hbm, dstv, wv, degp, acc16, rbuf, dbuf):
    c = lax.axis_index("c")
    s = lax.axis_index("s")
    wid = c * NS + s

    @pl.loop(0, ND // 16)
    def _zero(i):
        degp[pl.ds(i * 16, 16)] = jnp.zeros((16,), jnp.float32)

    pltpu.sync_copy(dst_hbm.at[wid], dstv)
    pltpu.sync_copy(w_hbm.at[wid], wv)

    @pl.loop(0, NCH)
    def _edges(j):
        for q in range(C // 16):
            idx = dstv[j, pl.ds(q * 16, 16)]
            vals = jnp.abs(wv[j, pl.ds(q * 16, 16)])
            plsc.addupdate_scatter(degp, [idx], vals)

    pltpu.sync_copy(degp, acc16.at[s])
    plsc.subcore_barrier()
    pltpu.sync_copy(acc16.at[:, pl.ds(s * DCH, DCH)], rbuf)

    @pl.loop(0, DCH // 16)
    def _red(q):
        tot = rbuf[0, pl.ds(q * 16, 16)]
        for r in range(1, NS):
            tot = tot + rbuf[r, pl.ds(q * 16, 16)]
        dbuf[pl.ds(q * 16, 16)] = tot

    pltpu.sync_copy(dbuf, out_hbm.at[c, pl.ds(s * DCH, DCH)])


_deg_call = functools.partial(
    pl.kernel,
    _deg_body,
    out_type=jax.ShapeDtypeStruct((NC, ND), jnp.float32),
    mesh=_mesh,
    scratch_types=[
        pltpu.VMEM((NCH, C), jnp.int32),
        pltpu.VMEM((NCH, C), jnp.float32),
        pltpu.VMEM((ND,), jnp.float32),
        pltpu.VMEM_SHARED((NS, ND), jnp.float32),
        pltpu.VMEM((NS, DCH), jnp.float32),
        pltpu.VMEM((DCH,), jnp.float32),
    ],
    compiler_params=_sc_params_nl,
)()


def _prop_body(g_hbm, src_hbm, dst_hbm, w_hbm, out_hbm, acc, srcv, dstv, wv,
               rows, zbuf, sems):
    c = lax.axis_index("c")
    s = lax.axis_index("s")
    wid = c * NS + s

    @pl.loop(0, ZB)
    def _zero(i):
        for q in range(D_H // 16):
            zbuf[i, pl.ds(q * 16, 16)] = jnp.zeros((16,), jnp.float32)

    @pl.when(s < NS - 1)
    def _z_full():
        for k in range(ZR // ZB):
            pltpu.sync_copy(zbuf, acc.at[pl.ds(s * ZR + k * ZB, ZB)])

    @pl.when(s == NS - 1)
    def _z_tail():
        for k in range(ZTAIL // ZB):
            pltpu.sync_copy(zbuf, acc.at[pl.ds((NS - 1) * ZR + k * ZB, ZB)])

    pltpu.sync_copy(src_hbm.at[wid], srcv)
    pltpu.sync_copy(dst_hbm.at[wid], dstv)
    pltpu.sync_copy(w_hbm.at[wid], wv)

    plsc.subcore_barrier()

    def _fire_gather(j, b):
        pltpu.make_async_copy(g_hbm.at[srcv.at[j]], rows.at[b],
                              sems.at[b]).start()

    def _drain_gather(b):
        pltpu.make_async_copy(g_hbm.at[srcv.at[0]], rows.at[b],
                              sems.at[b]).wait()

    def _fire_scatter(j, b):
        pltpu.async_copy(rows.at[b], acc.at[dstv.at[j]], sems.at[b], add=True)

    def _drain_scatter(b):
        pltpu.make_async_copy(rows.at[b], acc.at[dstv.at[0]],
                              sems.at[b]).wait()

    def _scale(j, b, unroll=5):
        @pl.loop(0, C // 16, unroll=unroll)
        def _grp(g):
            wvec = jnp.abs(wv[j, pl.ds(g * 16, 16)])
            for k in range(16):
                w = wvec[k]
                for q in range(D_H // 16):
                    e = g * 16 + k
                    rows[b, e, pl.ds(q * 16, 16)] = (
                        rows[b, e, pl.ds(q * 16, 16)] * w)

    for b in range(GD):
        _fire_gather(b, b)

    @pl.loop(0, MAIN)
    def _main(i):
        for b in range(NBUF):
            jv = i * NBUF + b
            b2 = (b + GD) % NBUF

            @pl.when(jv >= NBUF - GD)
            def _ds():
                _drain_scatter(b2)

            _fire_gather(jv + GD, b2)
            _drain_gather(b)
            _scale(jv, b)
            _fire_scatter(jv, b)

    for j in range(MAIN * NBUF, NCH):
        b = j % NBUF
        b2 = (j + GD) % NBUF
        if j >= NBUF - GD:
            _drain_scatter(b2)
        if j + GD < NCH:
            _fire_gather(j + GD, b2)
        _drain_gather(b)
        _scale(j, b, unroll=1)
        _fire_scatter(j, b)
    for j in range(NCH - (NBUF - GD), NCH):
        _drain_scatter(j % NBUF)

    plsc.subcore_barrier()

    @pl.when(s < NS - 1)
    def _wb_full():
        pltpu.sync_copy(acc.at[pl.ds(s * ZR, ZR)],
                        out_hbm.at[c, pl.ds(s * ZR, ZR)])

    @pl.when(s == NS - 1)
    def _wb_tail():
        pltpu.sync_copy(acc.at[pl.ds((NS - 1) * ZR, ZTAIL)],
                        out_hbm.at[c, pl.ds((NS - 1) * ZR, ZTAIL)])


_prop_call = functools.partial(
    pl.kernel,
    _prop_body,
    out_type=jax.ShapeDtypeStruct((NC, N, D_H), jnp.float32),
    mesh=_mesh,
    scratch_types=[
        pltpu.VMEM_SHARED((N, D_H), jnp.float32),
        pltpu.VMEM((NCH, C), jnp.int32),
        pltpu.VMEM((NCH, C), jnp.int32),
        pltpu.VMEM((NCH, C), jnp.float32),
        pltpu.VMEM((NBUF, C, D_H), jnp.float32),
        pltpu.VMEM((ZB, D_H), jnp.float32),
        pltpu.SemaphoreType.DMA((NBUF,)),
    ],
    compiler_params=_sc_params,
)()


def _k0_body(x_ref, w0_ref, degp_ref, g0_ref, dis_ref):
    dp = degp_ref[...]
    disr = lax.rsqrt(dp[0:1, :] + dp[1:2, :] + 1.0)
    dis_ref[...] = disr
    disc = jnp.transpose(disr, (1, 0))
    z = jnp.dot(x_ref[...], w0_ref[...], preferred_element_type=jnp.float32)
    g0_ref[...] = z * disc


def _k0_call(x, w0, degp):
    return pl.pallas_call(
        _k0_body,
        grid=(GRID,),
        in_specs=[
            pl.BlockSpec((R, D_IN), lambda i: (i, 0)),
            pl.BlockSpec((D_IN, D_H), lambda i: (0, 0)),
            pl.BlockSpec((NC, R), lambda i: (0, i)),
        ],
        out_specs=[
            pl.BlockSpec((R, D_H), lambda i: (i, 0)),
            pl.BlockSpec((1, R), lambda i: (0, i)),
        ],
        out_shape=[
            jax.ShapeDtypeStruct((N, D_H), jnp.float32),
            jax.ShapeDtypeStruct((1, ND), jnp.float32),
        ],
    )(x, w0, degp)


def _mid_body(pa_ref, pb_ref, g_ref, dis_ref, w_ref, b_ref, out_ref):
    disc = jnp.transpose(dis_ref[...], (1, 0))
    t = (pa_ref[0] + pb_ref[0] + g_ref[...]) * disc + b_ref[...]
    t = jnp.maximum(t * BN_SCALE, 0.0)
    out_ref[...] = jnp.dot(t, w_ref[...],
                           preferred_element_type=jnp.float32) * disc


def _mid_call(p, g, dis, w, b):
    return pl.pallas_call(
        _mid_body,
        grid=(GRID,),
        in_specs=[
            pl.BlockSpec((1, R, D_H), lambda i: (0, i, 0)),
            pl.BlockSpec((1, R, D_H), lambda i: (1, i, 0)),
            pl.BlockSpec((R, D_H), lambda i: (i, 0)),
            pl.BlockSpec((1, R), lambda i: (0, i)),
            pl.BlockSpec((D_H, D_H), lambda i: (0, 0)),
            pl.BlockSpec((1, D_H), lambda i: (0, 0)),
        ],
        out_specs=pl.BlockSpec((R, D_H), lambda i: (i, 0)),
        out_shape=jax.ShapeDtypeStruct((N, D_H), jnp.float32),
    )(p, p, g, dis, w, b)


def _fin_body(pa_ref, pb_ref, g_ref, dis_ref, b_ref, bat_ref, out_ref,
              ssum, cnt):
    i = pl.program_id(0)
    disc = jnp.transpose(dis_ref[...], (1, 0))
    t = (pa_ref[0] + pb_ref[0] + g_ref[...]) * disc + b_ref[...]
    t = jnp.maximum(t * BN_SCALE, 0.0)
    bat = bat_ref[...]
    lbl = lax.broadcasted_iota(jnp.int32, (NB, 1), 0)
    cidx = lax.broadcasted_iota(jnp.int32, (1, R), 1) + i * R
    ohT = jnp.where((bat == lbl) & (cidx < N), 1.0, 0.0)
    ps = lax.dot_general(ohT, t, (((1,), (0,)), ((), ())),
                         preferred_element_type=jnp.float32)
    pc = jnp.sum(ohT, axis=1)[:, None]

    @pl.when(i == 0)
    def _():
        ssum[...] = jnp.zeros_like(ssum)
        cnt[...] = jnp.zeros_like(cnt)

    ssum[...] += ps
    cnt[...] += pc
    mean = ssum[...] / jnp.maximum(cnt[...], 1.0)
    out_ref[...] = jnp.concatenate([mean, ssum[...]], axis=1)


def _fin_call(p, g, dis, b, bat):
    return pl.pallas_call(
        _fin_body,
        grid=(GRID,),
        in_specs=[
            pl.BlockSpec((1, R, D_H), lambda i: (0, i, 0)),
            pl.BlockSpec((1, R, D_H), lambda i: (1, i, 0)),
            pl.BlockSpec((R, D_H), lambda i: (i, 0)),
            pl.BlockSpec((1, R), lambda i: (0, i)),
            pl.BlockSpec((1, D_H), lambda i: (0, 0)),
            pl.BlockSpec((1, R), lambda i: (0, i)),
        ],
        out_specs=pl.BlockSpec((NB, 2 * D_H), lambda i: (0, 0)),
        out_shape=jax.ShapeDtypeStruct((NB, 2 * D_H), jnp.float32),
        scratch_shapes=[
            pltpu.VMEM((NB, D_H), jnp.float32),
            pltpu.VMEM((NB, 1), jnp.float32),
        ],
    )(p, p, g, dis, b, bat)


def kernel(x, edge_index, edge_weight, batch, W0, b0, W1, b1, W2, b2):
    src = edge_index[0].reshape(NW, NCH, C)
    dst = edge_index[1].reshape(NW, NCH, C)
    w2d = edge_weight.reshape(NW, NCH, C)
    bat2 = batch.reshape(1, N)

    degp = _deg_call(dst, w2d)
    g0, dis = _k0_call(x, W0, degp)

    p0 = _prop_call(g0, src, dst, w2d)
    g1 = _mid_call(p0, g0, dis, W1, b0.reshape(1, D_H))
    p1 = _prop_call(g1, src, dst, w2d)
    g2 = _mid_call(p1, g1, dis, W2, b1.reshape(1, D_H))
    p2 = _prop_call(g2, src, dst, w2d)
    return _fin_call(p2, g2, dis, b2.reshape(1, D_H), bat2)


# speedup vs baseline: 1.0293x; 1.0293x over previous
import functools

import jax
import jax.numpy as jnp
from jax import lax
from jax.experimental import pallas as pl
from jax.experimental.pallas import tpu as pltpu
from jax.experimental.pallas import tpu_sc as plsc

N = 10000
E = 320000
NB = 16
D_IN = 128
D_H = 64

NC = 2
NS = 16
NW = NC * NS
EPT = E // NW
C = 80
NCH = EPT // C
ZR = 640
ZTAIL = N - (NS - 1) * ZR
ZB = 80
NBUF = 10
ND = 10240

R = 1024
GRID = ND // R
DCH = ND // NS

BN_SCALE = float(1.0 / (1.0 + 1e-5) ** 0.5)

_mesh = plsc.VectorSubcoreMesh(core_axis_name="c", subcore_axis_name="s")
_sc_params = pltpu.CompilerParams(use_tc_tiling_on_sc=False)
_sc_params_nl = pltpu.CompilerParams(use_tc_tiling_on_sc=False,
                                     needs_layout_passes=False)


def _deg_body(dst_hbm, w_hbm, out_hbm, dstv, wv, degp, acc16, rbuf, dbuf):
    c = lax.axis_index("c")
    s = lax.axis_index("s")
    wid = c * NS + s

    @pl.loop(0, ND // 16)
    def _zero(i):
        degp[pl.ds(i * 16, 16)] = jnp.zeros((16,), jnp.float32)

    pltpu.sync_copy(dst_hbm.at[wid], dstv)
    pltpu.sync_copy(w_hbm.at[wid], wv)

    @pl.loop(0, NCH)
    def _edges(j):
        for q in range(C // 16):
            idx = dstv[j, pl.ds(q * 16, 16)]
            vals = jnp.abs(wv[j, pl.ds(q * 16, 16)])
            plsc.addupdate_scatter(degp, [idx], vals)

    pltpu.sync_copy(degp, acc16.at[s])
    plsc.subcore_barrier()
    pltpu.sync_copy(acc16.at[:, pl.ds(s * DCH, DCH)], rbuf)

    @pl.loop(0, DCH // 16)
    def _red(q):
        tot = rbuf[0, pl.ds(q * 16, 16)]
        for r in range(1, NS):
            tot = tot + rbuf[r, pl.ds(q * 16, 16)]
        dbuf[pl.ds(q * 16, 16)] = tot

    pltpu.sync_copy(dbuf, out_hbm.at[c, pl.ds(s * DCH, DCH)])


_deg_call = functools.partial(
    pl.kernel,
    _deg_body,
    out_type=jax.ShapeDtypeStruct((NC, ND), jnp.float32),
    mesh=_mesh,
    scratch_types=[
        pltpu.VMEM((NCH, C), jnp.int32),
        pltpu.VMEM((NCH, C), jnp.float32),
        pltpu.VMEM((ND,), jnp.float32),
        pltpu.VMEM_SHARED((NS, ND), jnp.float32),
        pltpu.VMEM((NS, DCH), jnp.float32),
        pltpu.VMEM((DCH,), jnp.float32),
    ],
    compiler_params=_sc_params_nl,
)()


def _prop_body(g_hbm, src_hbm, dst_hbm, w_hbm, out_hbm, acc, srcv, dstv, wv,
               rows, zbuf, sems):
    c = lax.axis_index("c")
    s = lax.axis_index("s")
    wid = c * NS + s

    @pl.loop(0, ZB)
    def _zero(i):
        for q in range(D_H // 16):
            zbuf[i, pl.ds(q * 16, 16)] = jnp.zeros((16,), jnp.float32)

    @pl.when(s < NS - 1)
    def _z_full():
        for k in range(ZR // ZB):
            pltpu.sync_copy(zbuf, acc.at[pl.ds(s * ZR + k * ZB, ZB)])

    @pl.when(s == NS - 1)
    def _z_tail():
        for k in range(ZTAIL // ZB):
            pltpu.sync_copy(zbuf, acc.at[pl.ds((NS - 1) * ZR + k * ZB, ZB)])

    pltpu.sync_copy(src_hbm.at[wid], srcv)
    pltpu.sync_copy(dst_hbm.at[wid], dstv)
    pltpu.sync_copy(w_hbm.at[wid], wv)

    plsc.subcore_barrier()

    def _fire_gather(j, b):
        pltpu.make_async_copy(g_hbm.at[srcv.at[j]], rows.at[b],
                              sems.at[b]).start()

    def _drain_gather(b):
        pltpu.make_async_copy(g_hbm.at[srcv.at[0]], rows.at[b],
                              sems.at[b]).wait()

    def _fire_scatter(j, b):
        pltpu.async_copy(rows.at[b], acc.at[dstv.at[j]], sems.at[b], add=True)

    def _drain_scatter(b):
        pltpu.make_async_copy(rows.at[b], acc.at[dstv.at[0]],
                              sems.at[b]).wait()

    def _scale(j, b):
        @pl.loop(0, C // 16, unroll=5)
        def _grp(g):
            wvec = jnp.abs(wv[j, pl.ds(g * 16, 16)])
            for k in range(16):
                w = wvec[k]
                for q in range(D_H // 16):
                    e = g * 16 + k
                    rows[b, e, pl.ds(q * 16, 16)] = (
                        rows[b, e, pl.ds(q * 16, 16)] * w)

    for b in range(NBUF // 2):
        _fire_gather(b, b)

    @pl.loop(0, (NCH - NBUF // 2) // NBUF)
    def _main(i):
        for b in range(NBUF):
            jv = i * NBUF + b
            b2 = (b + NBUF // 2) % NBUF
            jn = jv + NBUF // 2

            @pl.when(jv >= NBUF // 2)
            def _ds():
                _drain_scatter(b2)

            _fire_gather(jn, b2)
            _drain_gather(b)
            _scale(jv, b)
            _fire_scatter(jv, b)

    for j in range(NCH - NBUF // 2, NCH):
        b = j % NBUF
        _drain_gather(b)
        _scale(j, b)
        _fire_scatter(j, b)
    for j in range(NCH - NBUF, NCH):
        _drain_scatter(j % NBUF)

    plsc.subcore_barrier()

    @pl.when(s < NS - 1)
    def _wb_full():
        pltpu.sync_copy(acc.at[pl.ds(s * ZR, ZR)],
                        out_hbm.at[c, pl.ds(s * ZR, ZR)])

    @pl.when(s == NS - 1)
    def _wb_tail():
        pltpu.sync_copy(acc.at[pl.ds((NS - 1) * ZR, ZTAIL)],
                        out_hbm.at[c, pl.ds((NS - 1) * ZR, ZTAIL)])


_prop_call = functools.partial(
    pl.kernel,
    _prop_body,
    out_type=jax.ShapeDtypeStruct((NC, N, D_H), jnp.float32),
    mesh=_mesh,
    scratch_types=[
        pltpu.VMEM_SHARED((N, D_H), jnp.float32),
        pltpu.VMEM((NCH, C), jnp.int32),
        pltpu.VMEM((NCH, C), jnp.int32),
        pltpu.VMEM((NCH, C), jnp.float32),
        pltpu.VMEM((NBUF, C, D_H), jnp.float32),
        pltpu.VMEM((ZB, D_H), jnp.float32),
        pltpu.SemaphoreType.DMA((NBUF,)),
    ],
    compiler_params=_sc_params,
)()


def _k0_body(x_ref, w0_ref, degp_ref, g0_ref, dis_ref):
    dp = degp_ref[...]
    disr = lax.rsqrt(dp[0:1, :] + dp[1:2, :] + 1.0)
    dis_ref[...] = disr
    disc = jnp.transpose(disr, (1, 0))
    z = jnp.dot(x_ref[...], w0_ref[...], preferred_element_type=jnp.float32)
    g0_ref[...] = z * disc


def _k0_call(x, w0, degp):
    return pl.pallas_call(
        _k0_body,
        grid=(GRID,),
        in_specs=[
            pl.BlockSpec((R, D_IN), lambda i: (i, 0)),
            pl.BlockSpec((D_IN, D_H), lambda i: (0, 0)),
            pl.BlockSpec((NC, R), lambda i: (0, i)),
        ],
        out_specs=[
            pl.BlockSpec((R, D_H), lambda i: (i, 0)),
            pl.BlockSpec((1, R), lambda i: (0, i)),
        ],
        out_shape=[
            jax.ShapeDtypeStruct((N, D_H), jnp.float32),
            jax.ShapeDtypeStruct((1, ND), jnp.float32),
        ],
    )(x, w0, degp)


def _mid_body(pa_ref, pb_ref, g_ref, dis_ref, w_ref, b_ref, out_ref):
    disc = jnp.transpose(dis_ref[...], (1, 0))
    t = (pa_ref[0] + pb_ref[0] + g_ref[...]) * disc + b_ref[...]
    t = jnp.maximum(t * BN_SCALE, 0.0)
    out_ref[...] = jnp.dot(t, w_ref[...],
                           preferred_element_type=jnp.float32) * disc


def _mid_call(p, g, dis, w, b):
    return pl.pallas_call(
        _mid_body,
        grid=(GRID,),
        in_specs=[
            pl.BlockSpec((1, R, D_H), lambda i: (0, i, 0)),
            pl.BlockSpec((1, R, D_H), lambda i: (1, i, 0)),
            pl.BlockSpec((R, D_H), lambda i: (i, 0)),
            pl.BlockSpec((1, R), lambda i: (0, i)),
            pl.BlockSpec((D_H, D_H), lambda i: (0, 0)),
            pl.BlockSpec((1, D_H), lambda i: (0, 0)),
        ],
        out_specs=pl.BlockSpec((R, D_H), lambda i: (i, 0)),
        out_shape=jax.ShapeDtypeStruct((N, D_H), jnp.float32),
    )(p, p, g, dis, w, b)


def _fin_body(pa_ref, pb_ref, g_ref, dis_ref, b_ref, bat_ref, out_ref,
              ssum, cnt):
    i = pl.program_id(0)
    disc = jnp.transpose(dis_ref[...], (1, 0))
    t = (pa_ref[0] + pb_ref[0] + g_ref[...]) * disc + b_ref[...]
    t = jnp.maximum(t * BN_SCALE, 0.0)
    bat = bat_ref[...]
    lbl = lax.broadcasted_iota(jnp.int32, (NB, 1), 0)
    cidx = lax.broadcasted_iota(jnp.int32, (1, R), 1) + i * R
    ohT = jnp.where((bat == lbl) & (cidx < N), 1.0, 0.0)
    ps = lax.dot_general(ohT, t, (((1,), (0,)), ((), ())),
                         preferred_element_type=jnp.float32)
    pc = jnp.sum(ohT, axis=1)[:, None]

    @pl.when(i == 0)
    def _():
        ssum[...] = jnp.zeros_like(ssum)
        cnt[...] = jnp.zeros_like(cnt)

    ssum[...] += ps
    cnt[...] += pc
    mean = ssum[...] / jnp.maximum(cnt[...], 1.0)
    out_ref[...] = jnp.concatenate([mean, ssum[...]], axis=1)


def _fin_call(p, g, dis, b, bat):
    return pl.pallas_call(
        _fin_body,
        grid=(GRID,),
        in_specs=[
            pl.BlockSpec((1, R, D_H), lambda i: (0, i, 0)),
            pl.BlockSpec((1, R, D_H), lambda i: (1, i, 0)),
            pl.BlockSpec((R, D_H), lambda i: (i, 0)),
            pl.BlockSpec((1, R), lambda i: (0, i)),
            pl.BlockSpec((1, D_H), lambda i: (0, 0)),
            pl.BlockSpec((1, R), lambda i: (0, i)),
        ],
        out_specs=pl.BlockSpec((NB, 2 * D_H), lambda i: (0, 0)),
        out_shape=jax.ShapeDtypeStruct((NB, 2 * D_H), jnp.float32),
        scratch_shapes=[
            pltpu.VMEM((NB, D_H), jnp.float32),
            pltpu.VMEM((NB, 1), jnp.float32),
        ],
    )(p, p, g, dis, b, bat)


def kernel(x, edge_index, edge_weight, batch, W0, b0, W1, b1, W2, b2):
    src = edge_index[0].reshape(NW, NCH, C)
    dst = edge_index[1].reshape(NW, NCH, C)
    w2d = edge_weight.reshape(NW, NCH, C)
    bat2 = batch.reshape(1, N)

    degp = _deg_call(dst, w2d)
    g0, dis = _k0_call(x, W0, degp)

    p0 = _prop_call(g0, src, dst, w2d)
    g1 = _mid_call(p0, g0, dis, W1, b0.reshape(1, D_H))
    p1 = _prop_call(g1, src, dst, w2d)
    g2 = _mid_call(p1, g1, dis, W2, b1.reshape(1, D_H))
    p2 = _prop_call(g2, src, dst, w2d)
    return _fin_call(p2, g2, dis, b2.reshape(1, D_H), bat2)
